# Initial kernel scaffold; baseline (speedup 1.0000x reference)
import jax
import jax.numpy as jnp
from jax.experimental import pallas as pl


def kernel(semantic_logits, center_heatmap, offset_map, thing_class_ids):
    raise NotImplementedError("write your pallas kernel here")


# speedup vs baseline: 1.5964x; 1.5964x over previous
import functools

import jax
import jax.numpy as jnp
from jax.experimental import pallas as pl
from jax.experimental.pallas import tpu as pltpu

_THRESH = 0.1
_KEEP = 200
_NSLOT = 256
_DIV = 256.0
_VOID = 255 * 256
_AREA = 4096.0
_POISON = 1e30


def _iota(shape, dim):
    return jax.lax.broadcasted_iota(jnp.int32, shape, dim).astype(jnp.float32)


def _k1_body(logits_ref, hpad_ref, sem_ref, nms_ref, *, bh, C, W):
    rb = pl.program_id(1)
    best = logits_ref[0, 0]
    besti = jnp.zeros_like(best)
    for c in range(1, C):
        cur = logits_ref[0, c]
        upd = cur > best
        besti = jnp.where(upd, jnp.float32(c), besti)
        best = jnp.where(upd, cur, best)
    sem_ref[0] = besti.astype(jnp.int32)

    ht = hpad_ref[0, pl.ds(rb * bh, bh + 8), :]
    ht = jnp.where(ht > _THRESH, ht, jnp.zeros_like(ht))
    vert = ht[0:bh, :]
    for dr in range(1, 9):
        vert = jnp.maximum(vert, ht[dr:dr + bh, :])
    pooled = vert[:, 0:W]
    for dc in range(1, 9):
        pooled = jnp.maximum(pooled, vert[:, dc:dc + W])
    center = ht[4:4 + bh, 4:4 + W]
    nms_ref[0] = jnp.where(center == pooled, center, jnp.zeros_like(center))


def _k2_body(nms_ref, ys_ref, xs_ref, vl_ref, key_ref, *, H, W):
    nv = nms_ref[0]
    pos = nv > 0.0
    n_valid = jnp.sum(pos.astype(jnp.float32))
    flat = _iota((H, W), 0) * W + _iota((H, W), 1)
    key_b = jnp.where(pos, jnp.float32(H * W) - flat, jnp.zeros_like(nv))
    key = jnp.where(n_valid > jnp.float32(_KEEP), nv, key_b)
    key_ref[...] = key

    io_s = _iota((H, 1), 0)
    io_l = _iota((1, W), 1)
    io_k = _iota((1, _NSLOT), 1)
    rm0 = jnp.max(key, axis=1, keepdims=True)

    def body(k, carry):
        rm, ys, xs, vl = carry
        m = jnp.max(rm)
        valid = m > 0.0
        r_f = jnp.min(jnp.where(rm == m, io_s, jnp.float32(H)))
        ri = r_f.astype(jnp.int32)
        row = key_ref[pl.ds(ri, 1), :]
        c_f = jnp.min(jnp.where(row == m, io_l, jnp.float32(W)))
        newrow = jnp.where(io_l == c_f, jnp.zeros_like(row), row)
        key_ref[pl.ds(ri, 1), :] = newrow
        rm = jnp.where(io_s == r_f, jnp.max(newrow), rm)
        kf = k.astype(jnp.float32)
        slot = io_k == kf
        ys = jnp.where(slot, jnp.where(valid, r_f, _POISON), ys)
        xs = jnp.where(slot, jnp.where(valid, c_f, _POISON), xs)
        vl = jnp.where(slot, jnp.where(valid, 1.0, 0.0), vl)
        return rm, ys, xs, vl

    poison = jnp.full((1, _NSLOT), _POISON, jnp.float32)
    zeros = jnp.zeros((1, _NSLOT), jnp.float32)
    _, ys, xs, vl = jax.lax.fori_loop(0, _KEEP, body, (rm0, poison, poison, zeros))
    ys_ref[0] = ys
    xs_ref[0] = xs
    vl_ref[0] = vl


def _k3_body(off_ref, sem_ref, ys_ref, xs_ref, vl_ref, tid_ref,
             inst_ref, counts_ref, cacc_ref, *, bh, C, W, nrb, n_tid):
    rb = pl.program_id(1)
    ty = jnp.float32(bh) * rb.astype(jnp.float32) + _iota((bh, W), 0) + off_ref[0, 0]
    tx = _iota((bh, W), 1) + off_ref[0, 1]
    pp = ty * ty + tx * tx
    tyb = ty.astype(jnp.bfloat16).astype(jnp.float32)
    txb = tx.astype(jnp.bfloat16).astype(jnp.float32)
    ysv = ys_ref[0]
    xsv = xs_ref[0]
    has_v = jnp.max(vl_ref[0]) > 0.0
    io_k = _iota((1, _NSLOT), 1)

    def body(k, carry):
        best, bidx = carry
        kf = k.astype(jnp.float32)
        sel = io_k == kf
        cy = jnp.sum(jnp.where(sel, ysv, jnp.zeros_like(ysv)))
        cx = jnp.sum(jnp.where(sel, xsv, jnp.zeros_like(xsv)))
        cc = cy * cy + cx * cx
        cyb = cy.astype(jnp.bfloat16).astype(jnp.float32)
        cxb = cx.astype(jnp.bfloat16).astype(jnp.float32)
        dot = tyb * cyb + txb * cxb
        d = (pp + cc) - 2.0 * dot
        upd = d < best
        return jnp.where(upd, d, best), jnp.where(upd, kf, bidx)

    inf = jnp.full((bh, W), jnp.inf, jnp.float32)
    zero = jnp.zeros((bh, W), jnp.float32)
    _, bidx = jax.lax.fori_loop(0, _KEEP, body, (inf, zero))

    semv = sem_ref[0]
    thing = jnp.zeros_like(semv, dtype=jnp.bool_)
    for j in range(n_tid):
        thing = thing | (semv == tid_ref[0, 0, j])
    instf = jnp.where(thing & has_v, bidx + 1.0, jnp.zeros_like(bidx))
    inst_ref[0] = instf.astype(jnp.int32)

    sem_oh = (semv[:, None, :] == jax.lax.broadcasted_iota(
        jnp.int32, (1, C, 1), 1)).astype(jnp.float32)
    inst_oh = (instf[:, :, None] == _iota((1, 1, _NSLOT), 2)
               ).astype(jnp.float32)
    part = jax.lax.dot_general(
        sem_oh, inst_oh,
        dimension_numbers=(((2,), (1,)), ((0,), (0,))),
        preferred_element_type=jnp.float32)
    psum = jnp.sum(part, axis=0)

    @pl.when(rb == 0)
    def _():
        cacc_ref[...] = jnp.zeros((C, _NSLOT), jnp.float32)

    cacc_ref[...] += psum

    @pl.when(rb == nrb - 1)
    def _():
        counts_ref[0] = cacc_ref[...]


def _k4_body(sem_ref, inst_ref, counts_ref, tid_ref, pan_ref, *, bh, C, W, n_tid):
    cval = counts_ref[0]
    colmax = jnp.max(cval, axis=0, keepdims=True)
    ci = _iota((C, _NSLOT), 0)
    maj = jnp.min(jnp.where(cval == colmax, ci, jnp.float32(C)),
                  axis=0, keepdims=True)
    io_k = _iota((1, _NSLOT), 1)
    instv = inst_ref[0].astype(jnp.float32)

    def body(k, mpix):
        kf = k.astype(jnp.float32)
        mk = jnp.sum(jnp.where(io_k == kf, maj, jnp.zeros_like(maj)))
        return jnp.where(instv == kf, mk, mpix)

    mpix = jax.lax.fori_loop(1, _KEEP + 1, body, jnp.zeros((bh, W), jnp.float32))

    semv = sem_ref[0]
    areas = cval[:, 0:1]
    tv = tid_ref[0]
    io_cs = _iota((C, 1), 0)
    pan = jnp.full((bh, W), jnp.float32(_VOID), jnp.float32)
    for c in range(C):
        is_thing = jnp.max(jnp.where(tv == c, 1.0, 0.0)) > 0.0
        area_c = jnp.sum(jnp.where(io_cs == jnp.float32(c), areas,
                                   jnp.zeros_like(areas)))
        ok = jnp.logical_and(jnp.logical_not(is_thing), area_c >= _AREA)
        val = jnp.where(ok, jnp.float32(c) * _DIV, jnp.float32(_VOID))
        pan = jnp.where(semv == c, val, pan)
    pan = jnp.where(instv > 0.0, mpix * _DIV + instv, pan)
    pan_ref[0] = pan.astype(jnp.int32)


@functools.partial(jax.jit, static_argnames=())
def kernel(semantic_logits, center_heatmap, offset_map, thing_class_ids):
    B, C, H, W = semantic_logits.shape
    n_tid = thing_class_ids.shape[0]
    tids = thing_class_ids.reshape(1, 1, n_tid)

    bh1 = 64
    hpad = jnp.pad(center_heatmap[:, 0], ((0, 0), (4, 4), (4, 4)))
    sem, nms = pl.pallas_call(
        functools.partial(_k1_body, bh=bh1, C=C, W=W),
        grid=(B, H // bh1),
        in_specs=[
            pl.BlockSpec((1, C, bh1, W), lambda b, r: (b, 0, r, 0)),
            pl.BlockSpec((1, H + 8, W + 8), lambda b, r: (b, 0, 0)),
        ],
        out_specs=[
            pl.BlockSpec((1, bh1, W), lambda b, r: (b, r, 0)),
            pl.BlockSpec((1, bh1, W), lambda b, r: (b, r, 0)),
        ],
        out_shape=[
            jax.ShapeDtypeStruct((B, H, W), jnp.int32),
            jax.ShapeDtypeStruct((B, H, W), jnp.float32),
        ],
    )(semantic_logits, hpad)

    ys, xs, vl = pl.pallas_call(
        functools.partial(_k2_body, H=H, W=W),
        grid=(B,),
        in_specs=[pl.BlockSpec((1, H, W), lambda b: (b, 0, 0))],
        out_specs=[pl.BlockSpec((1, 1, _NSLOT), lambda b: (b, 0, 0))] * 3,
        out_shape=[jax.ShapeDtypeStruct((B, 1, _NSLOT), jnp.float32)] * 3,
        scratch_shapes=[pltpu.VMEM((H, W), jnp.float32)],
    )(nms)

    bh3 = 8
    nrb3 = H // bh3
    inst, counts = pl.pallas_call(
        functools.partial(_k3_body, bh=bh3, C=C, W=W, nrb=nrb3, n_tid=n_tid),
        grid=(B, nrb3),
        in_specs=[
            pl.BlockSpec((1, 2, bh3, W), lambda b, r: (b, 0, r, 0)),
            pl.BlockSpec((1, bh3, W), lambda b, r: (b, r, 0)),
            pl.BlockSpec((1, 1, _NSLOT), lambda b, r: (b, 0, 0)),
            pl.BlockSpec((1, 1, _NSLOT), lambda b, r: (b, 0, 0)),
            pl.BlockSpec((1, 1, _NSLOT), lambda b, r: (b, 0, 0)),
            pl.BlockSpec((1, 1, n_tid), lambda b, r: (0, 0, 0)),
        ],
        out_specs=[
            pl.BlockSpec((1, bh3, W), lambda b, r: (b, r, 0)),
            pl.BlockSpec((1, C, _NSLOT), lambda b, r: (b, 0, 0)),
        ],
        out_shape=[
            jax.ShapeDtypeStruct((B, H, W), jnp.int32),
            jax.ShapeDtypeStruct((B, C, _NSLOT), jnp.float32),
        ],
        scratch_shapes=[pltpu.VMEM((C, _NSLOT), jnp.float32)],
    )(offset_map, sem, ys, xs, vl, tids)

    bh4 = 64
    pan = pl.pallas_call(
        functools.partial(_k4_body, bh=bh4, C=C, W=W, n_tid=n_tid),
        grid=(B, H // bh4),
        in_specs=[
            pl.BlockSpec((1, bh4, W), lambda b, r: (b, r, 0)),
            pl.BlockSpec((1, bh4, W), lambda b, r: (b, r, 0)),
            pl.BlockSpec((1, C, _NSLOT), lambda b, r: (b, 0, 0)),
            pl.BlockSpec((1, 1, n_tid), lambda b, r: (0, 0, 0)),
        ],
        out_specs=pl.BlockSpec((1, bh4, W), lambda b, r: (b, r, 0)),
        out_shape=jax.ShapeDtypeStruct((B, H, W), jnp.int32),
    )(sem, inst, counts, tids)

    return pan, inst


# speedup vs baseline: 3.9530x; 2.4762x over previous
import functools

import jax
import jax.numpy as jnp
from jax.experimental import pallas as pl
from jax.experimental.pallas import tpu as pltpu

_THRESH = 0.1
_KEEP = 200
_NSLOT = 256
_DIV = 256.0
_VOID = 255 * 256
_AREA = 4096.0
_POISON = 1e30


def _iota(shape, dim):
    return jax.lax.broadcasted_iota(jnp.int32, shape, dim).astype(jnp.float32)


def _k1_body(logits_ref, hpad_ref, sem_ref, nms_ref, *, bh, C, W):
    rb = pl.program_id(1)
    best = logits_ref[0, 0]
    besti = jnp.zeros_like(best)
    for c in range(1, C):
        cur = logits_ref[0, c]
        upd = cur > best
        besti = jnp.where(upd, jnp.float32(c), besti)
        best = jnp.where(upd, cur, best)
    sem_ref[0] = besti.astype(jnp.int32)

    ht = hpad_ref[0, pl.ds(rb * bh, bh + 8), :]
    ht = jnp.where(ht > _THRESH, ht, jnp.zeros_like(ht))
    vert = ht[0:bh, :]
    for dr in range(1, 9):
        vert = jnp.maximum(vert, ht[dr:dr + bh, :])
    pooled = vert[:, 0:W]
    for dc in range(1, 9):
        pooled = jnp.maximum(pooled, vert[:, dc:dc + W])
    center = ht[4:4 + bh, 4:4 + W]
    nms_ref[0] = jnp.where(center == pooled, center, jnp.zeros_like(center))


def _k2_body(nms_ref, ys_ref, xs_ref, vl_ref, key_ref, *, H, W):
    nv = nms_ref[0]
    pos = nv > 0.0
    n_valid = jnp.sum(pos.astype(jnp.float32))
    flat = _iota((H, W), 0) * W + _iota((H, W), 1)
    key_b = jnp.where(pos, jnp.float32(H * W) - flat, jnp.zeros_like(nv))
    key = jnp.where(n_valid > jnp.float32(_KEEP), nv, key_b)
    key_ref[...] = key

    io_s = _iota((H, 1), 0)
    io_l = _iota((1, W), 1)
    io_k = _iota((1, _NSLOT), 1)
    rm0 = jnp.max(key, axis=1, keepdims=True)

    def body(k, carry):
        rm, ys, xs, vl = carry
        m = jnp.max(rm)
        valid = m > 0.0
        r_f = jnp.min(jnp.where(rm == m, io_s, jnp.float32(H)))
        ri = r_f.astype(jnp.int32)
        row = key_ref[pl.ds(ri, 1), :]
        c_f = jnp.min(jnp.where(row == m, io_l, jnp.float32(W)))
        newrow = jnp.where(io_l == c_f, jnp.zeros_like(row), row)
        key_ref[pl.ds(ri, 1), :] = newrow
        rm = jnp.where(io_s == r_f, jnp.max(newrow), rm)
        kf = k.astype(jnp.float32)
        slot = io_k == kf
        ys = jnp.where(slot, jnp.where(valid, r_f, _POISON), ys)
        xs = jnp.where(slot, jnp.where(valid, c_f, _POISON), xs)
        vl = jnp.where(slot, jnp.where(valid, 1.0, 0.0), vl)
        return rm, ys, xs, vl

    poison = jnp.full((1, _NSLOT), _POISON, jnp.float32)
    zeros = jnp.zeros((1, _NSLOT), jnp.float32)
    _, ys, xs, vl = jax.lax.fori_loop(0, _KEEP, body, (rm0, poison, poison, zeros))
    ys_ref[0] = ys
    xs_ref[0] = xs
    vl_ref[0] = vl


def _k3_body(off_ref, sem_ref, ys_ref, xs_ref, vl_ref, tid_ref,
             inst_ref, counts_ref, cacc_ref, *, bh, C, W, nrb, n_tid):
    rb = pl.program_id(1)
    ty = jnp.float32(bh) * rb.astype(jnp.float32) + _iota((bh, W), 0) + off_ref[0, 0]
    tx = _iota((bh, W), 1) + off_ref[0, 1]
    pp = ty * ty + tx * tx
    tyb = ty.astype(jnp.bfloat16).astype(jnp.float32)
    txb = tx.astype(jnp.bfloat16).astype(jnp.float32)
    ysv = ys_ref[0]
    xsv = xs_ref[0]
    has_v = jnp.max(vl_ref[0]) > 0.0
    io_k = _iota((1, _NSLOT), 1)

    def body(k, carry):
        best, bidx = carry
        kf = k.astype(jnp.float32)
        sel = io_k == kf
        cy = jnp.sum(jnp.where(sel, ysv, jnp.zeros_like(ysv)))
        cx = jnp.sum(jnp.where(sel, xsv, jnp.zeros_like(xsv)))
        cc = cy * cy + cx * cx
        cyb = cy.astype(jnp.bfloat16).astype(jnp.float32)
        cxb = cx.astype(jnp.bfloat16).astype(jnp.float32)
        dot = tyb * cyb + txb * cxb
        d = (pp + cc) - 2.0 * dot
        upd = d < best
        return jnp.where(upd, d, best), jnp.where(upd, kf, bidx)

    inf = jnp.full((bh, W), jnp.inf, jnp.float32)
    zero = jnp.zeros((bh, W), jnp.float32)
    _, bidx = jax.lax.fori_loop(0, _KEEP, body, (inf, zero))

    semv = sem_ref[0]
    thing = jnp.zeros_like(semv, dtype=jnp.bool_)
    for j in range(n_tid):
        thing = thing | (semv == tid_ref[0, 0, j])
    instf = jnp.where(thing & has_v, bidx + 1.0, jnp.zeros_like(bidx))
    inst_ref[0] = instf.astype(jnp.int32)

    sem_oh = (semv[:, None, :] == jax.lax.broadcasted_iota(
        jnp.int32, (1, C, 1), 1)).astype(jnp.float32)
    inst_oh = (instf[:, :, None] == _iota((1, 1, _NSLOT), 2)
               ).astype(jnp.float32)
    part = jax.lax.dot_general(
        sem_oh, inst_oh,
        dimension_numbers=(((2,), (1,)), ((0,), (0,))),
        preferred_element_type=jnp.float32)
    psum = jnp.sum(part, axis=0)

    @pl.when(rb == 0)
    def _():
        cacc_ref[...] = jnp.zeros((C, _NSLOT), jnp.float32)

    cacc_ref[...] += psum

    @pl.when(rb == nrb - 1)
    def _():
        counts_ref[0] = cacc_ref[...]


def _k4_body(sem_ref, inst_ref, counts_ref, tid_ref, pan_ref, *, bh, C, W, n_tid):
    cval = counts_ref[0]
    colmax = jnp.max(cval, axis=0, keepdims=True)
    ci = _iota((C, _NSLOT), 0)
    maj = jnp.min(jnp.where(cval == colmax, ci, jnp.float32(C)),
                  axis=0, keepdims=True)
    io_k = _iota((1, _NSLOT), 1)
    instv = inst_ref[0].astype(jnp.float32)

    def body(k, mpix):
        kf = k.astype(jnp.float32)
        mk = jnp.sum(jnp.where(io_k == kf, maj, jnp.zeros_like(maj)))
        return jnp.where(instv == kf, mk, mpix)

    mpix = jax.lax.fori_loop(1, _KEEP + 1, body, jnp.zeros((bh, W), jnp.float32))

    semv = sem_ref[0]
    areas = cval[:, 0:1]
    tv = tid_ref[0]
    io_cs = _iota((C, 1), 0)
    pan = jnp.full((bh, W), jnp.float32(_VOID), jnp.float32)
    for c in range(C):
        is_thing = jnp.max(jnp.where(tv == c, 1.0, 0.0)) > 0.0
        area_c = jnp.sum(jnp.where(io_cs == jnp.float32(c), areas,
                                   jnp.zeros_like(areas)))
        ok = jnp.logical_and(jnp.logical_not(is_thing), area_c >= _AREA)
        val = jnp.where(ok, jnp.float32(c) * _DIV, jnp.float32(_VOID))
        pan = jnp.where(semv == c, val, pan)
    pan = jnp.where(instv > 0.0, mpix * _DIV + instv, pan)
    pan_ref[0] = pan.astype(jnp.int32)


@functools.partial(jax.jit, static_argnames=())
def kernel(semantic_logits, center_heatmap, offset_map, thing_class_ids):
    B, C, H, W = semantic_logits.shape
    n_tid = thing_class_ids.shape[0]
    tids = thing_class_ids.reshape(1, 1, n_tid)

    bh1 = 64
    hpad = jnp.pad(center_heatmap[:, 0], ((0, 0), (4, 4), (4, 4)))
    sem, nms = pl.pallas_call(
        functools.partial(_k1_body, bh=bh1, C=C, W=W),
        grid=(B, H // bh1),
        in_specs=[
            pl.BlockSpec((1, C, bh1, W), lambda b, r: (b, 0, r, 0)),
            pl.BlockSpec((1, H + 8, W + 8), lambda b, r: (b, 0, 0)),
        ],
        out_specs=[
            pl.BlockSpec((1, bh1, W), lambda b, r: (b, r, 0)),
            pl.BlockSpec((1, bh1, W), lambda b, r: (b, r, 0)),
        ],
        out_shape=[
            jax.ShapeDtypeStruct((B, H, W), jnp.int32),
            jax.ShapeDtypeStruct((B, H, W), jnp.float32),
        ],
    )(semantic_logits, hpad)

    ys, xs, vl = pl.pallas_call(
        functools.partial(_k2_body, H=H, W=W),
        grid=(B,),
        in_specs=[pl.BlockSpec((1, H, W), lambda b: (b, 0, 0))],
        out_specs=[pl.BlockSpec((1, 1, _NSLOT), lambda b: (b, 0, 0))] * 3,
        out_shape=[jax.ShapeDtypeStruct((B, 1, _NSLOT), jnp.float32)] * 3,
        scratch_shapes=[pltpu.VMEM((H, W), jnp.float32)],
    )(nms)

    bh3 = 32
    nrb3 = H // bh3
    inst, counts = pl.pallas_call(
        functools.partial(_k3_body, bh=bh3, C=C, W=W, nrb=nrb3, n_tid=n_tid),
        grid=(B, nrb3),
        in_specs=[
            pl.BlockSpec((1, 2, bh3, W), lambda b, r: (b, 0, r, 0)),
            pl.BlockSpec((1, bh3, W), lambda b, r: (b, r, 0)),
            pl.BlockSpec((1, 1, _NSLOT), lambda b, r: (b, 0, 0)),
            pl.BlockSpec((1, 1, _NSLOT), lambda b, r: (b, 0, 0)),
            pl.BlockSpec((1, 1, _NSLOT), lambda b, r: (b, 0, 0)),
            pl.BlockSpec((1, 1, n_tid), lambda b, r: (0, 0, 0)),
        ],
        out_specs=[
            pl.BlockSpec((1, bh3, W), lambda b, r: (b, r, 0)),
            pl.BlockSpec((1, C, _NSLOT), lambda b, r: (b, 0, 0)),
        ],
        out_shape=[
            jax.ShapeDtypeStruct((B, H, W), jnp.int32),
            jax.ShapeDtypeStruct((B, C, _NSLOT), jnp.float32),
        ],
        scratch_shapes=[pltpu.VMEM((C, _NSLOT), jnp.float32)],
    )(offset_map, sem, ys, xs, vl, tids)

    bh4 = 64
    pan = pl.pallas_call(
        functools.partial(_k4_body, bh=bh4, C=C, W=W, n_tid=n_tid),
        grid=(B, H // bh4),
        in_specs=[
            pl.BlockSpec((1, bh4, W), lambda b, r: (b, r, 0)),
            pl.BlockSpec((1, bh4, W), lambda b, r: (b, r, 0)),
            pl.BlockSpec((1, C, _NSLOT), lambda b, r: (b, 0, 0)),
            pl.BlockSpec((1, 1, n_tid), lambda b, r: (0, 0, 0)),
        ],
        out_specs=pl.BlockSpec((1, bh4, W), lambda b, r: (b, r, 0)),
        out_shape=jax.ShapeDtypeStruct((B, H, W), jnp.int32),
    )(sem, inst, counts, tids)

    return pan, inst


# speedup vs baseline: 4.9342x; 1.2482x over previous
import functools

import jax
import jax.numpy as jnp
from jax.experimental import pallas as pl
from jax.experimental.pallas import tpu as pltpu

_THRESH = 0.1
_KEEP = 200
_NSLOT = 256
_DIV = 256.0
_VOID = 255 * 256
_AREA = 4096.0
_POISON = 1e30


def _iota(shape, dim):
    return jax.lax.broadcasted_iota(jnp.int32, shape, dim).astype(jnp.float32)


def _k1_body(logits_ref, hpad_ref, sem_ref, nms_ref, *, bh, C, W):
    rb = pl.program_id(1)
    best = logits_ref[0, 0]
    besti = jnp.zeros_like(best)
    for c in range(1, C):
        cur = logits_ref[0, c]
        upd = cur > best
        besti = jnp.where(upd, jnp.float32(c), besti)
        best = jnp.where(upd, cur, best)
    sem_ref[0] = besti.astype(jnp.int32)

    ht = hpad_ref[0, pl.ds(rb * bh, bh + 8), :]
    ht = jnp.where(ht > _THRESH, ht, jnp.zeros_like(ht))
    vert = ht[0:bh, :]
    for dr in range(1, 9):
        vert = jnp.maximum(vert, ht[dr:dr + bh, :])
    pooled = vert[:, 0:W]
    for dc in range(1, 9):
        pooled = jnp.maximum(pooled, vert[:, dc:dc + W])
    center = ht[4:4 + bh, 4:4 + W]
    nms_ref[0] = jnp.where(center == pooled, center, jnp.zeros_like(center))


def _k2_body(nms_ref, ys_ref, xs_ref, vl_ref, key_ref, *, H, W):
    nv = nms_ref[0]
    pos = nv > 0.0
    n_valid = jnp.sum(pos.astype(jnp.float32))
    flat = _iota((H, W), 0) * W + _iota((H, W), 1)
    key_b = jnp.where(pos, jnp.float32(H * W) - flat, jnp.zeros_like(nv))
    key = jnp.where(n_valid > jnp.float32(_KEEP), nv, key_b)
    key_ref[...] = key

    io_s = _iota((H, 1), 0)
    io_l = _iota((1, W), 1)
    io_k = _iota((1, _NSLOT), 1)
    rm0 = jnp.max(key, axis=1, keepdims=True)

    def body(k, carry):
        rm, ys, xs, vl = carry
        m = jnp.max(rm)
        valid = m > 0.0
        r_f = jnp.min(jnp.where(rm == m, io_s, jnp.float32(H)))
        ri = r_f.astype(jnp.int32)
        row = key_ref[pl.ds(ri, 1), :]
        c_f = jnp.min(jnp.where(row == m, io_l, jnp.float32(W)))
        newrow = jnp.where(io_l == c_f, jnp.zeros_like(row), row)
        key_ref[pl.ds(ri, 1), :] = newrow
        rm = jnp.where(io_s == r_f, jnp.max(newrow), rm)
        kf = k.astype(jnp.float32)
        slot = io_k == kf
        ys = jnp.where(slot, jnp.where(valid, r_f, _POISON), ys)
        xs = jnp.where(slot, jnp.where(valid, c_f, _POISON), xs)
        vl = jnp.where(slot, jnp.where(valid, 1.0, 0.0), vl)
        return rm, ys, xs, vl

    poison = jnp.full((1, _NSLOT), _POISON, jnp.float32)
    zeros = jnp.zeros((1, _NSLOT), jnp.float32)
    _, ys, xs, vl = jax.lax.fori_loop(0, _KEEP, body, (rm0, poison, poison, zeros))
    ys_ref[0] = ys
    xs_ref[0] = xs
    vl_ref[0] = vl


def _k3_body(off_ref, sem_ref, ys_ref, xs_ref, vl_ref, tid_ref,
             inst_ref, counts_ref, cacc_ref, *, bh, C, W, nrb, n_tid):
    rb = pl.program_id(1)
    ty = jnp.float32(bh) * rb.astype(jnp.float32) + _iota((bh, W), 0) + off_ref[0, 0]
    tx = _iota((bh, W), 1) + off_ref[0, 1]
    pp = ty * ty + tx * tx
    tyb = ty.astype(jnp.bfloat16).astype(jnp.float32)
    txb = tx.astype(jnp.bfloat16).astype(jnp.float32)
    ysv = ys_ref[0]
    xsv = xs_ref[0]
    has_v = jnp.max(vl_ref[0]) > 0.0
    io_k = _iota((1, _NSLOT), 1)

    def body(k, carry):
        best, bidx = carry
        kf = k.astype(jnp.float32)
        sel = io_k == kf
        cy = jnp.sum(jnp.where(sel, ysv, jnp.zeros_like(ysv)))
        cx = jnp.sum(jnp.where(sel, xsv, jnp.zeros_like(xsv)))
        cc = cy * cy + cx * cx
        cyb = cy.astype(jnp.bfloat16).astype(jnp.float32)
        cxb = cx.astype(jnp.bfloat16).astype(jnp.float32)
        dot = tyb * cyb + txb * cxb
        d = (pp + cc) - 2.0 * dot
        upd = d < best
        return jnp.where(upd, d, best), jnp.where(upd, kf, bidx)

    inf = jnp.full((bh, W), jnp.inf, jnp.float32)
    zero = jnp.zeros((bh, W), jnp.float32)
    _, bidx = jax.lax.fori_loop(0, _KEEP, body, (inf, zero))

    semv = sem_ref[0]
    thing = jnp.zeros_like(semv, dtype=jnp.bool_)
    for j in range(n_tid):
        thing = thing | (semv == tid_ref[0, 0, j])
    instf = jnp.where(thing & has_v, bidx + 1.0, jnp.zeros_like(bidx))
    inst_ref[0] = instf.astype(jnp.int32)

    sem_oh = (semv[:, None, :] == jax.lax.broadcasted_iota(
        jnp.int32, (1, C, 1), 1)).astype(jnp.float32)
    inst_oh = (instf[:, :, None] == _iota((1, 1, _NSLOT), 2)
               ).astype(jnp.float32)
    part = jax.lax.dot_general(
        sem_oh, inst_oh,
        dimension_numbers=(((2,), (1,)), ((0,), (0,))),
        preferred_element_type=jnp.float32)
    psum = jnp.sum(part, axis=0)

    @pl.when(rb == 0)
    def _():
        cacc_ref[...] = jnp.zeros((C, _NSLOT), jnp.float32)

    cacc_ref[...] += psum

    @pl.when(rb == nrb - 1)
    def _():
        counts_ref[0] = cacc_ref[...]


def _k4_body(sem_ref, inst_ref, counts_ref, tid_ref, pan_ref, *, bh, C, W, n_tid):
    cval = counts_ref[0]
    colmax = jnp.max(cval, axis=0, keepdims=True)
    ci = _iota((C, _NSLOT), 0)
    maj = jnp.min(jnp.where(cval == colmax, ci, jnp.float32(C)),
                  axis=0, keepdims=True)
    io_k = _iota((1, _NSLOT), 1)
    instv = inst_ref[0].astype(jnp.float32)

    def body(k, mpix):
        kf = k.astype(jnp.float32)
        mk = jnp.sum(jnp.where(io_k == kf, maj, jnp.zeros_like(maj)))
        return jnp.where(instv == kf, mk, mpix)

    mpix = jax.lax.fori_loop(1, _KEEP + 1, body, jnp.zeros((bh, W), jnp.float32))

    semv = sem_ref[0]
    areas = cval[:, 0:1]
    tv = tid_ref[0]
    io_cs = _iota((C, 1), 0)
    pan = jnp.full((bh, W), jnp.float32(_VOID), jnp.float32)
    for c in range(C):
        is_thing = jnp.max(jnp.where(tv == c, 1.0, 0.0)) > 0.0
        area_c = jnp.sum(jnp.where(io_cs == jnp.float32(c), areas,
                                   jnp.zeros_like(areas)))
        ok = jnp.logical_and(jnp.logical_not(is_thing), area_c >= _AREA)
        val = jnp.where(ok, jnp.float32(c) * _DIV, jnp.float32(_VOID))
        pan = jnp.where(semv == c, val, pan)
    pan = jnp.where(instv > 0.0, mpix * _DIV + instv, pan)
    pan_ref[0] = pan.astype(jnp.int32)


@functools.partial(jax.jit, static_argnames=())
def kernel(semantic_logits, center_heatmap, offset_map, thing_class_ids):
    B, C, H, W = semantic_logits.shape
    n_tid = thing_class_ids.shape[0]
    tids = thing_class_ids.reshape(1, 1, n_tid)

    bh1 = 64
    hpad = jnp.pad(center_heatmap[:, 0], ((0, 0), (4, 4), (4, 4)))
    sem, nms = pl.pallas_call(
        functools.partial(_k1_body, bh=bh1, C=C, W=W),
        grid=(B, H // bh1),
        in_specs=[
            pl.BlockSpec((1, C, bh1, W), lambda b, r: (b, 0, r, 0)),
            pl.BlockSpec((1, H + 8, W + 8), lambda b, r: (b, 0, 0)),
        ],
        out_specs=[
            pl.BlockSpec((1, bh1, W), lambda b, r: (b, r, 0)),
            pl.BlockSpec((1, bh1, W), lambda b, r: (b, r, 0)),
        ],
        out_shape=[
            jax.ShapeDtypeStruct((B, H, W), jnp.int32),
            jax.ShapeDtypeStruct((B, H, W), jnp.float32),
        ],
    )(semantic_logits, hpad)

    ys, xs, vl = pl.pallas_call(
        functools.partial(_k2_body, H=H, W=W),
        grid=(B,),
        in_specs=[pl.BlockSpec((1, H, W), lambda b: (b, 0, 0))],
        out_specs=[pl.BlockSpec((1, 1, _NSLOT), lambda b: (b, 0, 0))] * 3,
        out_shape=[jax.ShapeDtypeStruct((B, 1, _NSLOT), jnp.float32)] * 3,
        scratch_shapes=[pltpu.VMEM((H, W), jnp.float32)],
    )(nms)

    bh3 = 64
    nrb3 = H // bh3
    inst, counts = pl.pallas_call(
        functools.partial(_k3_body, bh=bh3, C=C, W=W, nrb=nrb3, n_tid=n_tid),
        grid=(B, nrb3),
        in_specs=[
            pl.BlockSpec((1, 2, bh3, W), lambda b, r: (b, 0, r, 0)),
            pl.BlockSpec((1, bh3, W), lambda b, r: (b, r, 0)),
            pl.BlockSpec((1, 1, _NSLOT), lambda b, r: (b, 0, 0)),
            pl.BlockSpec((1, 1, _NSLOT), lambda b, r: (b, 0, 0)),
            pl.BlockSpec((1, 1, _NSLOT), lambda b, r: (b, 0, 0)),
            pl.BlockSpec((1, 1, n_tid), lambda b, r: (0, 0, 0)),
        ],
        out_specs=[
            pl.BlockSpec((1, bh3, W), lambda b, r: (b, r, 0)),
            pl.BlockSpec((1, C, _NSLOT), lambda b, r: (b, 0, 0)),
        ],
        out_shape=[
            jax.ShapeDtypeStruct((B, H, W), jnp.int32),
            jax.ShapeDtypeStruct((B, C, _NSLOT), jnp.float32),
        ],
        scratch_shapes=[pltpu.VMEM((C, _NSLOT), jnp.float32)],
    )(offset_map, sem, ys, xs, vl, tids)

    bh4 = 64
    pan = pl.pallas_call(
        functools.partial(_k4_body, bh=bh4, C=C, W=W, n_tid=n_tid),
        grid=(B, H // bh4),
        in_specs=[
            pl.BlockSpec((1, bh4, W), lambda b, r: (b, r, 0)),
            pl.BlockSpec((1, bh4, W), lambda b, r: (b, r, 0)),
            pl.BlockSpec((1, C, _NSLOT), lambda b, r: (b, 0, 0)),
            pl.BlockSpec((1, 1, n_tid), lambda b, r: (0, 0, 0)),
        ],
        out_specs=pl.BlockSpec((1, bh4, W), lambda b, r: (b, r, 0)),
        out_shape=jax.ShapeDtypeStruct((B, H, W), jnp.int32),
    )(sem, inst, counts, tids)

    return pan, inst
